# baseline (speedup 1.0000x reference)
import jax
import jax.numpy as jnp
from jax import lax
from jax.experimental import pallas as pl
from jax.experimental.pallas import tpu as pltpu
from jax.experimental.pallas import tpu_sc as plsc

B = 16384
D = 64
W = D // 2
NC = 2
NS = 16
L = 16
NW = NC * NS
BPW = B // NW
CHUNK = 128
NPASS = BPW // CHUNK
CGROUPS = CHUNK // L


def _bpr_body(users_hbm, pos_hbm, neg_hbm, uemb_hbm, iemb_hbm, ibias_hbm,
              out_hbm,
              uidx_v, pidx_v, nidx_v, urows_v, prows_v, nrows_v,
              pb_v, nb_v, out_v,
              sem_rows, sem_bias):
    wid = lax.axis_index("s") * NC + lax.axis_index("c")
    base = wid * BPW

    pltpu.sync_copy(users_hbm.at[pl.ds(base, BPW)], uidx_v)
    pltpu.sync_copy(pos_hbm.at[pl.ds(base, BPW)], pidx_v)
    pltpu.sync_copy(neg_hbm.at[pl.ds(base, BPW)], nidx_v)

    def fire(p):
        slot = p % 2
        off = p * CHUNK
        ur = urows_v.at[slot]
        pr = prows_v.at[slot]
        nr = nrows_v.at[slot]

        def fire_group(g, carry):
            gb = off + g * L
            uvec = uidx_v[pl.ds(gb, L)]
            pvec = pidx_v[pl.ds(gb, L)]
            nvec = nidx_v[pl.ds(gb, L)]
            for jj in range(L):
                j = g * L + jj
                pltpu.async_copy(uemb_hbm.at[uvec[jj]], ur.at[j],
                                 sem_rows.at[slot])
                pltpu.async_copy(iemb_hbm.at[pvec[jj]], pr.at[j],
                                 sem_rows.at[slot])
                pltpu.async_copy(iemb_hbm.at[nvec[jj]], nr.at[j],
                                 sem_rows.at[slot])
            return carry

        lax.fori_loop(0, CGROUPS, fire_group, 0, unroll=False)
        cpb = pltpu.async_copy(
            ibias_hbm.at[pidx_v.at[pl.ds(off, CHUNK)]], pb_v.at[slot],
            sem_bias.at[slot])
        cnb = pltpu.async_copy(
            ibias_hbm.at[nidx_v.at[pl.ds(off, CHUNK)]], nb_v.at[slot],
            sem_bias.at[slot])
        return cpb, cnb

    def drain(p, cpb, cnb):
        slot = p % 2
        dummy = uemb_hbm.at[pl.ds(0, CHUNK)]
        pltpu.make_async_copy(dummy, urows_v.at[slot],
                              sem_rows.at[slot]).wait()
        pltpu.make_async_copy(dummy, prows_v.at[slot],
                              sem_rows.at[slot]).wait()
        pltpu.make_async_copy(dummy, nrows_v.at[slot],
                              sem_rows.at[slot]).wait()
        cpb.wait()
        cnb.wait()

    lane = lax.iota(jnp.int32, L)
    himask = jnp.full((L,), -65536, jnp.int32)

    def unpack(x):
        lo = plsc.bitcast(x << 16, jnp.float32)
        hi = plsc.bitcast(x & himask, jnp.float32)
        return lo, hi

    def compute(p):
        slot = p % 2
        off = p * CHUNK
        ur = urows_v.at[slot]
        pr = prows_v.at[slot]
        nr = nrows_v.at[slot]

        def group(g, carry):
            rows = g * L + lane
            acc = (pb_v[slot, pl.ds(g * L, L)]
                   - nb_v[slot, pl.ds(g * L, L)])
            for w in range(W):
                col = jnp.full((L,), w, jnp.int32)
                ulo, uhi = unpack(plsc.load_gather(ur, [rows, col]))
                plo, phi = unpack(plsc.load_gather(pr, [rows, col]))
                nlo, nhi = unpack(plsc.load_gather(nr, [rows, col]))
                acc = acc + ulo * (plo - nlo) + uhi * (phi - nhi)
            out_v[pl.ds(off + g * L, L)] = acc
            return carry

        lax.fori_loop(0, CGROUPS, group, 0, unroll=False)

    pending = fire(0)
    for p in range(NPASS):
        nxt = fire(p + 1) if p + 1 < NPASS else None
        drain(p, *pending)
        compute(p)
        pending = nxt

    pltpu.sync_copy(out_v, out_hbm.at[pl.ds(base, BPW)])


@jax.jit
def _bpr_call(users, pos_items, neg_items, uemb_packed, iemb_packed,
              item_biases_flat):
    mesh = plsc.VectorSubcoreMesh(core_axis_name="c", subcore_axis_name="s")
    return pl.kernel(
        _bpr_body,
        out_type=jax.ShapeDtypeStruct((B,), jnp.float32),
        mesh=mesh,
        compiler_params=pltpu.CompilerParams(
            needs_layout_passes=False, use_tc_tiling_on_sc=True),
        scratch_types=[
            pltpu.VMEM((BPW,), jnp.int32),
            pltpu.VMEM((BPW,), jnp.int32),
            pltpu.VMEM((BPW,), jnp.int32),
            pltpu.VMEM((2, CHUNK, W), jnp.int32),
            pltpu.VMEM((2, CHUNK, W), jnp.int32),
            pltpu.VMEM((2, CHUNK, W), jnp.int32),
            pltpu.VMEM((2, CHUNK), jnp.float32),
            pltpu.VMEM((2, CHUNK), jnp.float32),
            pltpu.VMEM((BPW,), jnp.float32),
            pltpu.SemaphoreType.DMA((2,)),
            pltpu.SemaphoreType.DMA((2,)),
        ],
    )(users, pos_items, neg_items, uemb_packed, iemb_packed,
      item_biases_flat)


def _pack(table):
    return jax.lax.bitcast_convert_type(
        table.astype(jnp.bfloat16).reshape(-1, W, 2), jnp.int32)


def kernel(users, pos_items, neg_items, user_embeddings, item_embeddings,
           user_biases, item_biases):
    del user_biases
    return _bpr_call(users, pos_items, neg_items, _pack(user_embeddings),
                     _pack(item_embeddings), item_biases.reshape(-1))


# speedup vs baseline: 1.6799x; 1.6799x over previous
import jax
import jax.numpy as jnp
from jax import lax
from jax.experimental import pallas as pl
from jax.experimental.pallas import tpu as pltpu
from jax.experimental.pallas import tpu_sc as plsc

B = 16384
D = 64
W = D // 2
NC = 2
NS = 16
L = 16
NW = NC * NS
BPW = B // NW
CHUNK = 128
NPASS = BPW // CHUNK
CGROUPS = CHUNK // L


def _bpr_body(users_hbm, pos_hbm, neg_hbm, uemb_hbm, iemb_hbm, ibias_hbm,
              out_hbm,
              uidx_v, pidx_v, nidx_v, urows_v, prows_v, nrows_v,
              pb_v, nb_v, out_v,
              sem_rows, sem_bias):
    wid = lax.axis_index("s") * NC + lax.axis_index("c")
    base = wid * BPW

    pltpu.sync_copy(users_hbm.at[pl.ds(base, BPW)], uidx_v)
    pltpu.sync_copy(pos_hbm.at[pl.ds(base, BPW)], pidx_v)
    pltpu.sync_copy(neg_hbm.at[pl.ds(base, BPW)], nidx_v)

    def fire(p):
        slot = p % 2
        off = p * CHUNK
        ur = urows_v.at[slot]
        pr = prows_v.at[slot]
        nr = nrows_v.at[slot]

        def fire_group(g, carry):
            gb = off + g * L
            uvec = uidx_v[pl.ds(gb, L)]
            pvec = pidx_v[pl.ds(gb, L)]
            nvec = nidx_v[pl.ds(gb, L)]
            for jj in range(L):
                j = g * L + jj
                pltpu.async_copy(uemb_hbm.at[uvec[jj]], ur.at[j],
                                 sem_rows.at[slot])
                pltpu.async_copy(iemb_hbm.at[pvec[jj]], pr.at[j],
                                 sem_rows.at[slot])
                pltpu.async_copy(iemb_hbm.at[nvec[jj]], nr.at[j],
                                 sem_rows.at[slot])
            return carry

        lax.fori_loop(0, CGROUPS, fire_group, 0, unroll=False)
        cpb = pltpu.async_copy(
            ibias_hbm.at[pidx_v.at[pl.ds(off, CHUNK)]], pb_v.at[slot],
            sem_bias.at[slot])
        cnb = pltpu.async_copy(
            ibias_hbm.at[nidx_v.at[pl.ds(off, CHUNK)]], nb_v.at[slot],
            sem_bias.at[slot])
        return cpb, cnb

    def drain(p, cpb, cnb):
        slot = p % 2
        dummy = uemb_hbm.at[pl.ds(0, CHUNK)]
        pltpu.make_async_copy(dummy, urows_v.at[slot],
                              sem_rows.at[slot]).wait()
        pltpu.make_async_copy(dummy, prows_v.at[slot],
                              sem_rows.at[slot]).wait()
        pltpu.make_async_copy(dummy, nrows_v.at[slot],
                              sem_rows.at[slot]).wait()
        cpb.wait()
        cnb.wait()

    lane = lax.iota(jnp.int32, L)
    himask = jnp.full((L,), -65536, jnp.int32)

    def unpack(x):
        lo = plsc.bitcast(x << 16, jnp.float32)
        hi = plsc.bitcast(x & himask, jnp.float32)
        return lo, hi

    def compute(p):
        slot = p % 2
        off = p * CHUNK
        ur = urows_v.at[slot]
        pr = prows_v.at[slot]
        nr = nrows_v.at[slot]

        def group(g, carry):
            rows = g * L + lane
            acc = (pb_v[slot, pl.ds(g * L, L)]
                   - nb_v[slot, pl.ds(g * L, L)])
            for w in range(W):
                col = jnp.full((L,), w, jnp.int32)
                ulo, uhi = unpack(plsc.load_gather(ur, [rows, col]))
                plo, phi = unpack(plsc.load_gather(pr, [rows, col]))
                nlo, nhi = unpack(plsc.load_gather(nr, [rows, col]))
                acc = acc + ulo * (plo - nlo) + uhi * (phi - nhi)
            out_v[pl.ds(off + g * L, L)] = acc
            return carry

        lax.fori_loop(0, CGROUPS, group, 0, unroll=False)

    pending = fire(0)
    for p in range(NPASS):
        nxt = fire(p + 1) if p + 1 < NPASS else None
        drain(p, *pending)
        compute(p)
        pending = nxt

    pltpu.sync_copy(out_v, out_hbm.at[pl.ds(base, BPW)])


@jax.jit
def _bpr_call(users, pos_items, neg_items, uemb_packed, iemb_packed,
              item_biases_flat):
    mesh = plsc.VectorSubcoreMesh(core_axis_name="c", subcore_axis_name="s")
    return pl.kernel(
        _bpr_body,
        out_type=jax.ShapeDtypeStruct((B,), jnp.float32),
        mesh=mesh,
        compiler_params=pltpu.CompilerParams(
            needs_layout_passes=False, use_tc_tiling_on_sc=True),
        scratch_types=[
            pltpu.VMEM((BPW,), jnp.int32),
            pltpu.VMEM((BPW,), jnp.int32),
            pltpu.VMEM((BPW,), jnp.int32),
            pltpu.VMEM((2, CHUNK, W), jnp.int32),
            pltpu.VMEM((2, CHUNK, W), jnp.int32),
            pltpu.VMEM((2, CHUNK, W), jnp.int32),
            pltpu.VMEM((2, CHUNK), jnp.float32),
            pltpu.VMEM((2, CHUNK), jnp.float32),
            pltpu.VMEM((BPW,), jnp.float32),
            pltpu.SemaphoreType.DMA((2,)),
            pltpu.SemaphoreType.DMA((2,)),
        ],
    )(users, pos_items, neg_items, uemb_packed, iemb_packed,
      item_biases_flat)


def _pack(table):
    a = lax.bitcast_convert_type(table[:, :W], jnp.uint32)
    b = lax.bitcast_convert_type(table[:, W:], jnp.uint32)

    def rne(u):
        return (u + jnp.uint32(0x7FFF) + ((u >> 16) & jnp.uint32(1))) >> 16

    packed = rne(a) | (rne(b) << 16)
    return lax.bitcast_convert_type(packed, jnp.int32)


def kernel(users, pos_items, neg_items, user_embeddings, item_embeddings,
           user_biases, item_biases):
    del user_biases
    return _bpr_call(users, pos_items, neg_items, _pack(user_embeddings),
                     _pack(item_embeddings), item_biases.reshape(-1))


# speedup vs baseline: 2.0152x; 1.1996x over previous
import jax
import jax.numpy as jnp
from jax import lax
from jax.experimental import pallas as pl
from jax.experimental.pallas import tpu as pltpu
from jax.experimental.pallas import tpu_sc as plsc

B = 16384
D = 64
W = D // 2
NC = 2
NS = 16
L = 16
NW = NC * NS
BPW = B // NW
CHUNK = 128
NPASS = BPW // CHUNK
CGROUPS = CHUNK // L


def _bpr_body(users_hbm, pos_hbm, neg_hbm, uemb_hbm, iemb_hbm, ibias_hbm,
              out_hbm,
              uidx_v, pidx_v, nidx_v, urows_v, prows_v, nrows_v,
              pb_v, nb_v, out_v,
              sem_rows, sem_bias):
    wid = lax.axis_index("s") * NC + lax.axis_index("c")
    base = wid * BPW

    pltpu.sync_copy(users_hbm.at[pl.ds(base, BPW)], uidx_v)
    pltpu.sync_copy(pos_hbm.at[pl.ds(base, BPW)], pidx_v)
    pltpu.sync_copy(neg_hbm.at[pl.ds(base, BPW)], nidx_v)

    def fire(p):
        slot = p % 2
        off = p * CHUNK
        ur = urows_v.at[slot]
        pr = prows_v.at[slot]
        nr = nrows_v.at[slot]

        def fire_group(g, carry):
            gb = off + g * L
            uvec = uidx_v[pl.ds(gb, L)]
            pvec = pidx_v[pl.ds(gb, L)]
            nvec = nidx_v[pl.ds(gb, L)]
            for jj in range(L):
                j = g * L + jj
                pltpu.async_copy(uemb_hbm.at[uvec[jj]], ur.at[j],
                                 sem_rows.at[slot])
                pltpu.async_copy(iemb_hbm.at[pvec[jj]], pr.at[j],
                                 sem_rows.at[slot])
                pltpu.async_copy(iemb_hbm.at[nvec[jj]], nr.at[j],
                                 sem_rows.at[slot])
            return carry

        lax.fori_loop(0, CGROUPS, fire_group, 0, unroll=False)
        cpb = pltpu.async_copy(
            ibias_hbm.at[pidx_v.at[pl.ds(off, CHUNK)]], pb_v.at[slot],
            sem_bias.at[slot])
        cnb = pltpu.async_copy(
            ibias_hbm.at[nidx_v.at[pl.ds(off, CHUNK)]], nb_v.at[slot],
            sem_bias.at[slot])
        return cpb, cnb

    def drain(p, cpb, cnb):
        slot = p % 2
        dummy = uemb_hbm.at[pl.ds(0, CHUNK)]
        pltpu.make_async_copy(dummy, urows_v.at[slot],
                              sem_rows.at[slot]).wait()
        pltpu.make_async_copy(dummy, prows_v.at[slot],
                              sem_rows.at[slot]).wait()
        pltpu.make_async_copy(dummy, nrows_v.at[slot],
                              sem_rows.at[slot]).wait()
        cpb.wait()
        cnb.wait()

    lane = lax.iota(jnp.int32, L)
    himask = jnp.full((L,), -65536, jnp.int32)

    def unpack(x):
        lo = plsc.bitcast(x << 16, jnp.float32)
        hi = plsc.bitcast(x & himask, jnp.float32)
        return lo, hi

    def compute(p):
        slot = p % 2
        off = p * CHUNK
        ur = urows_v.at[slot]
        pr = prows_v.at[slot]
        nr = nrows_v.at[slot]

        def group(g, carry):
            rows = g * L + lane
            acc = (pb_v[slot, pl.ds(g * L, L)]
                   - nb_v[slot, pl.ds(g * L, L)])
            for w in range(W):
                col = jnp.full((L,), w, jnp.int32)
                ulo, uhi = unpack(plsc.load_gather(ur, [rows, col]))
                plo, phi = unpack(plsc.load_gather(pr, [rows, col]))
                nlo, nhi = unpack(plsc.load_gather(nr, [rows, col]))
                acc = acc + ulo * (plo - nlo) + uhi * (phi - nhi)
            out_v[pl.ds(off + g * L, L)] = acc
            return carry

        lax.fori_loop(0, CGROUPS, group, 0, unroll=False)

    pending = fire(0)
    for p in range(NPASS):
        nxt = fire(p + 1) if p + 1 < NPASS else None
        drain(p, *pending)
        compute(p)
        pending = nxt

    pltpu.sync_copy(out_v, out_hbm.at[pl.ds(base, BPW)])


@jax.jit
def _bpr_call(users, pos_items, neg_items, uemb_packed, iemb_packed,
              item_biases_flat):
    mesh = plsc.VectorSubcoreMesh(core_axis_name="c", subcore_axis_name="s")
    return pl.kernel(
        _bpr_body,
        out_type=jax.ShapeDtypeStruct((B,), jnp.float32),
        mesh=mesh,
        compiler_params=pltpu.CompilerParams(
            needs_layout_passes=False, use_tc_tiling_on_sc=True),
        scratch_types=[
            pltpu.VMEM((BPW,), jnp.int32),
            pltpu.VMEM((BPW,), jnp.int32),
            pltpu.VMEM((BPW,), jnp.int32),
            pltpu.VMEM((2, CHUNK, W), jnp.int32),
            pltpu.VMEM((2, CHUNK, W), jnp.int32),
            pltpu.VMEM((2, CHUNK, W), jnp.int32),
            pltpu.VMEM((2, CHUNK), jnp.float32),
            pltpu.VMEM((2, CHUNK), jnp.float32),
            pltpu.VMEM((BPW,), jnp.float32),
            pltpu.SemaphoreType.DMA((2,)),
            pltpu.SemaphoreType.DMA((2,)),
        ],
    )(users, pos_items, neg_items, uemb_packed, iemb_packed,
      item_biases_flat)


_PACK_BLK = 1024


def _pack_body(tT_ref, out_ref):
    x = tT_ref[...]
    a = lax.bitcast_convert_type(x[:W, :], jnp.uint32)
    b = lax.bitcast_convert_type(x[W:, :], jnp.uint32)

    def rne(u):
        return (u + jnp.uint32(0x7FFF) + ((u >> 16) & jnp.uint32(1))) >> 16

    packed = rne(a) | (rne(b) << 16)
    out_ref[...] = lax.bitcast_convert_type(packed.T, jnp.int32)


def _pack(table_T):
    n = table_T.shape[1]
    grid = (n + _PACK_BLK - 1) // _PACK_BLK
    return pl.pallas_call(
        _pack_body,
        grid=(grid,),
        in_specs=[pl.BlockSpec((D, _PACK_BLK), lambda i: (0, i))],
        out_specs=pl.BlockSpec((_PACK_BLK, W), lambda i: (i, 0)),
        out_shape=jax.ShapeDtypeStruct((n, W), jnp.int32),
    )(table_T)


def kernel(users, pos_items, neg_items, user_embeddings, item_embeddings,
           user_biases, item_biases):
    del user_biases
    return _bpr_call(users, pos_items, neg_items, _pack(user_embeddings.T),
                     _pack(item_embeddings.T), item_biases.reshape(-1))


# speedup vs baseline: 3.9894x; 1.9796x over previous
import jax
import jax.numpy as jnp
from jax import lax
from jax.experimental import pallas as pl
from jax.experimental.pallas import tpu as pltpu
from jax.experimental.pallas import tpu_sc as plsc

B = 16384
D = 64
W = D // 2
NC = 2
NS = 16
L = 16
NW = NC * NS
BPW = B // NW
CHUNK = 128
NPASS = BPW // CHUNK
CGROUPS = CHUNK // L


def _bpr_body(users_hbm, pos_hbm, neg_hbm, uemb_hbm, iemb_hbm, ibias_hbm,
              out_hbm,
              uidx_v, pidx_v, nidx_v, urows_v, prows_v, nrows_v,
              pb_v, nb_v, out_v,
              sem_rows, sem_bias):
    wid = lax.axis_index("s") * NC + lax.axis_index("c")
    base = wid * BPW

    pltpu.sync_copy(users_hbm.at[pl.ds(base, BPW)], uidx_v)
    pltpu.sync_copy(pos_hbm.at[pl.ds(base, BPW)], pidx_v)
    pltpu.sync_copy(neg_hbm.at[pl.ds(base, BPW)], nidx_v)

    def fire(p):
        slot = p % 2
        off = p * CHUNK
        ur = urows_v.at[slot]
        pr = prows_v.at[slot]
        nr = nrows_v.at[slot]

        def fire_group(g, carry):
            gb = off + g * L
            uvec = uidx_v[pl.ds(gb, L)]
            pvec = pidx_v[pl.ds(gb, L)]
            nvec = nidx_v[pl.ds(gb, L)]
            for jj in range(L):
                j = g * L + jj
                pltpu.async_copy(uemb_hbm.at[uvec[jj]], ur.at[j],
                                 sem_rows.at[slot])
                pltpu.async_copy(iemb_hbm.at[pvec[jj]], pr.at[j],
                                 sem_rows.at[slot])
                pltpu.async_copy(iemb_hbm.at[nvec[jj]], nr.at[j],
                                 sem_rows.at[slot])
            return carry

        lax.fori_loop(0, CGROUPS, fire_group, 0, unroll=False)
        cpb = pltpu.async_copy(
            ibias_hbm.at[pidx_v.at[pl.ds(off, CHUNK)]], pb_v.at[slot],
            sem_bias.at[slot])
        cnb = pltpu.async_copy(
            ibias_hbm.at[nidx_v.at[pl.ds(off, CHUNK)]], nb_v.at[slot],
            sem_bias.at[slot])
        return cpb, cnb

    def drain(p, cpb, cnb):
        slot = p % 2
        dummy = uemb_hbm.at[pl.ds(0, CHUNK)]
        pltpu.make_async_copy(dummy, urows_v.at[slot],
                              sem_rows.at[slot]).wait()
        pltpu.make_async_copy(dummy, prows_v.at[slot],
                              sem_rows.at[slot]).wait()
        pltpu.make_async_copy(dummy, nrows_v.at[slot],
                              sem_rows.at[slot]).wait()
        cpb.wait()
        cnb.wait()

    lane = lax.iota(jnp.int32, L)
    himask = jnp.full((L,), -65536, jnp.int32)

    def unpack(x):
        lo = plsc.bitcast(x << 16, jnp.float32)
        hi = plsc.bitcast(x & himask, jnp.float32)
        return lo, hi

    def compute(p):
        slot = p % 2
        off = p * CHUNK
        ur = urows_v.at[slot]
        pr = prows_v.at[slot]
        nr = nrows_v.at[slot]

        def group(g, carry):
            rows = g * L + lane
            acc = (pb_v[slot, pl.ds(g * L, L)]
                   - nb_v[slot, pl.ds(g * L, L)])
            for w in range(W):
                col = jnp.full((L,), w, jnp.int32)
                ulo, uhi = unpack(plsc.load_gather(ur, [rows, col]))
                plo, phi = unpack(plsc.load_gather(pr, [rows, col]))
                nlo, nhi = unpack(plsc.load_gather(nr, [rows, col]))
                acc = acc + ulo * (plo - nlo) + uhi * (phi - nhi)
            out_v[pl.ds(off + g * L, L)] = acc
            return carry

        lax.fori_loop(0, CGROUPS, group, 0, unroll=False)

    pending = fire(0)
    for p in range(NPASS):
        nxt = fire(p + 1) if p + 1 < NPASS else None
        drain(p, *pending)
        compute(p)
        pending = nxt

    pltpu.sync_copy(out_v, out_hbm.at[pl.ds(base, BPW)])


@jax.jit
def _bpr_call(users, pos_items, neg_items, uemb_packed, iemb_packed,
              item_biases_flat):
    mesh = plsc.VectorSubcoreMesh(core_axis_name="c", subcore_axis_name="s")
    return pl.kernel(
        _bpr_body,
        out_type=jax.ShapeDtypeStruct((B,), jnp.float32),
        mesh=mesh,
        compiler_params=pltpu.CompilerParams(
            needs_layout_passes=False, use_tc_tiling_on_sc=True),
        scratch_types=[
            pltpu.VMEM((BPW,), jnp.int32),
            pltpu.VMEM((BPW,), jnp.int32),
            pltpu.VMEM((BPW,), jnp.int32),
            pltpu.VMEM((2, CHUNK, W), jnp.int32),
            pltpu.VMEM((2, CHUNK, W), jnp.int32),
            pltpu.VMEM((2, CHUNK, W), jnp.int32),
            pltpu.VMEM((2, CHUNK), jnp.float32),
            pltpu.VMEM((2, CHUNK), jnp.float32),
            pltpu.VMEM((BPW,), jnp.float32),
            pltpu.SemaphoreType.DMA((2,)),
            pltpu.SemaphoreType.DMA((2,)),
        ],
    )(users, pos_items, neg_items, uemb_packed, iemb_packed,
      item_biases_flat)


_PACK_BLK = 4096


def _pack_body(tT_ref, out_ref):
    x = tT_ref[...]
    a = lax.bitcast_convert_type(x[:W, :], jnp.uint32)
    b = lax.bitcast_convert_type(x[W:, :], jnp.uint32)

    def rne(u):
        return (u + jnp.uint32(0x7FFF) + ((u >> 16) & jnp.uint32(1))) >> 16

    packed = rne(a) | (rne(b) << 16)
    out_ref[...] = lax.bitcast_convert_type(packed.T, jnp.int32)


def _pack(table_T):
    n = table_T.shape[1]
    grid = (n + _PACK_BLK - 1) // _PACK_BLK
    return pl.pallas_call(
        _pack_body,
        grid=(grid,),
        in_specs=[pl.BlockSpec((D, _PACK_BLK), lambda i: (0, i))],
        out_specs=pl.BlockSpec((_PACK_BLK, W), lambda i: (i, 0)),
        out_shape=jax.ShapeDtypeStruct((n, W), jnp.int32),
    )(table_T)


def kernel(users, pos_items, neg_items, user_embeddings, item_embeddings,
           user_biases, item_biases):
    del user_biases
    return _bpr_call(users, pos_items, neg_items, _pack(user_embeddings.T),
                     _pack(item_embeddings.T), item_biases.reshape(-1))


# speedup vs baseline: 5.2791x; 1.3233x over previous
import jax
import jax.numpy as jnp
from jax import lax
from jax.experimental import pallas as pl
from jax.experimental.pallas import tpu as pltpu
from jax.experimental.pallas import tpu_sc as plsc

B = 16384
D = 64
W = D // 2
NC = 2
NS = 16
L = 16
NW = NC * NS
BPW = B // NW
CHUNK = 128
NPASS = BPW // CHUNK
CGROUPS = CHUNK // L


def _bpr_body(users_hbm, pos_hbm, neg_hbm, uemb_hbm, iemb_hbm, ibias_hbm,
              out_hbm,
              uidx_v, pidx_v, nidx_v, urows_v, prows_v, nrows_v,
              pb_v, nb_v, out_v,
              sem_rows, sem_bias):
    wid = lax.axis_index("s") * NC + lax.axis_index("c")
    base = wid * BPW

    pltpu.sync_copy(users_hbm.at[pl.ds(base, BPW)], uidx_v)
    pltpu.sync_copy(pos_hbm.at[pl.ds(base, BPW)], pidx_v)
    pltpu.sync_copy(neg_hbm.at[pl.ds(base, BPW)], nidx_v)

    def fire(p):
        slot = p % 2
        off = p * CHUNK
        ur = urows_v.at[slot]
        pr = prows_v.at[slot]
        nr = nrows_v.at[slot]

        def fire_group(g, carry):
            gb = off + g * L
            uvec = uidx_v[pl.ds(gb, L)]
            pvec = pidx_v[pl.ds(gb, L)]
            nvec = nidx_v[pl.ds(gb, L)]
            for jj in range(L):
                j = g * L + jj
                pltpu.async_copy(uemb_hbm.at[uvec[jj]], ur.at[j],
                                 sem_rows.at[slot])
                pltpu.async_copy(iemb_hbm.at[pvec[jj]], pr.at[j],
                                 sem_rows.at[slot])
                pltpu.async_copy(iemb_hbm.at[nvec[jj]], nr.at[j],
                                 sem_rows.at[slot])
            return carry

        lax.fori_loop(0, CGROUPS, fire_group, 0, unroll=False)
        cpb = pltpu.async_copy(
            ibias_hbm.at[pidx_v.at[pl.ds(off, CHUNK)]], pb_v.at[slot],
            sem_bias.at[slot])
        cnb = pltpu.async_copy(
            ibias_hbm.at[nidx_v.at[pl.ds(off, CHUNK)]], nb_v.at[slot],
            sem_bias.at[slot])
        return cpb, cnb

    def drain(p, cpb, cnb):
        slot = p % 2
        dummy = uemb_hbm.at[pl.ds(0, CHUNK)]
        pltpu.make_async_copy(dummy, urows_v.at[slot],
                              sem_rows.at[slot]).wait()
        pltpu.make_async_copy(dummy, prows_v.at[slot],
                              sem_rows.at[slot]).wait()
        pltpu.make_async_copy(dummy, nrows_v.at[slot],
                              sem_rows.at[slot]).wait()
        cpb.wait()
        cnb.wait()

    lane = lax.iota(jnp.int32, L)
    himask = jnp.full((L,), -65536, jnp.int32)

    def unpack(x):
        lo = plsc.bitcast(x << 16, jnp.float32)
        hi = plsc.bitcast(x & himask, jnp.float32)
        return lo, hi

    def compute(p):
        slot = p % 2
        off = p * CHUNK
        ur = urows_v.at[slot]
        pr = prows_v.at[slot]
        nr = nrows_v.at[slot]

        def group(g, carry):
            rows = g * L + lane
            acc = (pb_v[slot, pl.ds(g * L, L)]
                   - nb_v[slot, pl.ds(g * L, L)])
            for w in range(W):
                col = jnp.full((L,), w, jnp.int32)
                ulo, uhi = unpack(plsc.load_gather(ur, [rows, col]))
                plo, phi = unpack(plsc.load_gather(pr, [rows, col]))
                nlo, nhi = unpack(plsc.load_gather(nr, [rows, col]))
                acc = acc + ulo * (plo - nlo) + uhi * (phi - nhi)
            out_v[pl.ds(off + g * L, L)] = acc
            return carry

        lax.fori_loop(0, CGROUPS, group, 0, unroll=False)

    pending = fire(0)
    for p in range(NPASS):
        nxt = fire(p + 1) if p + 1 < NPASS else None
        drain(p, *pending)
        compute(p)
        pending = nxt

    pltpu.sync_copy(out_v, out_hbm.at[pl.ds(base, BPW)])


@jax.jit
def _bpr_call(users, pos_items, neg_items, uemb_packed, iemb_packed,
              item_biases_flat):
    mesh = plsc.VectorSubcoreMesh(core_axis_name="c", subcore_axis_name="s")
    return pl.kernel(
        _bpr_body,
        out_type=jax.ShapeDtypeStruct((B,), jnp.float32),
        mesh=mesh,
        compiler_params=pltpu.CompilerParams(
            needs_layout_passes=False, use_tc_tiling_on_sc=True),
        scratch_types=[
            pltpu.VMEM((BPW,), jnp.int32),
            pltpu.VMEM((BPW,), jnp.int32),
            pltpu.VMEM((BPW,), jnp.int32),
            pltpu.VMEM((2, CHUNK, W), jnp.int32),
            pltpu.VMEM((2, CHUNK, W), jnp.int32),
            pltpu.VMEM((2, CHUNK, W), jnp.int32),
            pltpu.VMEM((2, CHUNK), jnp.float32),
            pltpu.VMEM((2, CHUNK), jnp.float32),
            pltpu.VMEM((BPW,), jnp.float32),
            pltpu.SemaphoreType.DMA((2,)),
            pltpu.SemaphoreType.DMA((2,)),
        ],
    )(users, pos_items, neg_items, uemb_packed, iemb_packed,
      item_biases_flat)


_PACK_BLK = 16384


def _pack_body(tT_ref, out_ref):
    x = tT_ref[...]
    a = lax.bitcast_convert_type(x[:W, :], jnp.uint32)
    b = lax.bitcast_convert_type(x[W:, :], jnp.uint32)

    def rne(u):
        return (u + jnp.uint32(0x7FFF) + ((u >> 16) & jnp.uint32(1))) >> 16

    packed = rne(a) | (rne(b) << 16)
    out_ref[...] = lax.bitcast_convert_type(packed.T, jnp.int32)


def _pack(table_T):
    n = table_T.shape[1]
    grid = (n + _PACK_BLK - 1) // _PACK_BLK
    return pl.pallas_call(
        _pack_body,
        grid=(grid,),
        in_specs=[pl.BlockSpec((D, _PACK_BLK), lambda i: (0, i))],
        out_specs=pl.BlockSpec((_PACK_BLK, W), lambda i: (i, 0)),
        out_shape=jax.ShapeDtypeStruct((n, W), jnp.int32),
    )(table_T)


def kernel(users, pos_items, neg_items, user_embeddings, item_embeddings,
           user_biases, item_biases):
    del user_biases
    return _bpr_call(users, pos_items, neg_items, _pack(user_embeddings.T),
                     _pack(item_embeddings.T), item_biases.reshape(-1))


# speedup vs baseline: 5.3961x; 1.0222x over previous
import jax
import jax.numpy as jnp
from jax import lax
from jax.experimental import pallas as pl
from jax.experimental.pallas import tpu as pltpu
from jax.experimental.pallas import tpu_sc as plsc

B = 16384
D = 64
W = D // 2
NC = 2
NS = 16
L = 16
NW = NC * NS
BPW = B // NW
CHUNK = 128
NPASS = BPW // CHUNK
CGROUPS = CHUNK // L


def _bpr_body(users_hbm, pos_hbm, neg_hbm, uemb_hbm, iemb_hbm, ibias_hbm,
              out_hbm,
              uidx_v, pidx_v, nidx_v, urows_v, prows_v, nrows_v,
              pb_v, nb_v, out_v,
              sem_rows, sem_bias):
    wid = lax.axis_index("s") * NC + lax.axis_index("c")
    base = wid * BPW

    pltpu.sync_copy(users_hbm.at[pl.ds(base, BPW)], uidx_v)
    pltpu.sync_copy(pos_hbm.at[pl.ds(base, BPW)], pidx_v)
    pltpu.sync_copy(neg_hbm.at[pl.ds(base, BPW)], nidx_v)

    def fire(p):
        slot = p % 2
        off = p * CHUNK
        ur = urows_v.at[slot]
        pr = prows_v.at[slot]
        nr = nrows_v.at[slot]

        def fire_group(g, carry):
            gb = off + g * L
            uvec = uidx_v[pl.ds(gb, L)]
            pvec = pidx_v[pl.ds(gb, L)]
            nvec = nidx_v[pl.ds(gb, L)]
            for jj in range(L):
                j = g * L + jj
                pltpu.async_copy(uemb_hbm.at[uvec[jj]], ur.at[j],
                                 sem_rows.at[slot])
                pltpu.async_copy(iemb_hbm.at[pvec[jj]], pr.at[j],
                                 sem_rows.at[slot])
                pltpu.async_copy(iemb_hbm.at[nvec[jj]], nr.at[j],
                                 sem_rows.at[slot])
            return carry

        lax.fori_loop(0, CGROUPS, fire_group, 0, unroll=False)
        cpb = pltpu.async_copy(
            ibias_hbm.at[pidx_v.at[pl.ds(off, CHUNK)]], pb_v.at[slot],
            sem_bias.at[slot])
        cnb = pltpu.async_copy(
            ibias_hbm.at[nidx_v.at[pl.ds(off, CHUNK)]], nb_v.at[slot],
            sem_bias.at[slot])
        return cpb, cnb

    def drain(p, cpb, cnb):
        slot = p % 2
        dummy = uemb_hbm.at[pl.ds(0, CHUNK)]
        pltpu.make_async_copy(dummy, urows_v.at[slot],
                              sem_rows.at[slot]).wait()
        pltpu.make_async_copy(dummy, prows_v.at[slot],
                              sem_rows.at[slot]).wait()
        pltpu.make_async_copy(dummy, nrows_v.at[slot],
                              sem_rows.at[slot]).wait()
        cpb.wait()
        cnb.wait()

    lane = lax.iota(jnp.int32, L)
    himask = jnp.full((L,), -65536, jnp.int32)

    def unpack(x):
        lo = plsc.bitcast(x << 16, jnp.float32)
        hi = plsc.bitcast(x & himask, jnp.float32)
        return lo, hi

    def compute(p):
        slot = p % 2
        off = p * CHUNK
        ur = urows_v.at[slot]
        pr = prows_v.at[slot]
        nr = nrows_v.at[slot]

        def group(g, carry):
            rows = g * L + lane
            acc = (pb_v[slot, pl.ds(g * L, L)]
                   - nb_v[slot, pl.ds(g * L, L)])
            for w in range(W):
                col = jnp.full((L,), w, jnp.int32)
                ulo, uhi = unpack(plsc.load_gather(ur, [rows, col]))
                plo, phi = unpack(plsc.load_gather(pr, [rows, col]))
                nlo, nhi = unpack(plsc.load_gather(nr, [rows, col]))
                acc = acc + ulo * (plo - nlo) + uhi * (phi - nhi)
            out_v[pl.ds(off + g * L, L)] = acc
            return carry

        lax.fori_loop(0, CGROUPS, group, 0, unroll=False)

    pending = fire(0)
    for p in range(NPASS):
        nxt = fire(p + 1) if p + 1 < NPASS else None
        drain(p, *pending)
        compute(p)
        pending = nxt

    pltpu.sync_copy(out_v, out_hbm.at[pl.ds(base, BPW)])


@jax.jit
def _bpr_call(users, pos_items, neg_items, uemb_packed, iemb_packed,
              item_biases_flat):
    mesh = plsc.VectorSubcoreMesh(core_axis_name="c", subcore_axis_name="s")
    return pl.kernel(
        _bpr_body,
        out_type=jax.ShapeDtypeStruct((B,), jnp.float32),
        mesh=mesh,
        compiler_params=pltpu.CompilerParams(
            needs_layout_passes=False, use_tc_tiling_on_sc=True),
        scratch_types=[
            pltpu.VMEM((BPW,), jnp.int32),
            pltpu.VMEM((BPW,), jnp.int32),
            pltpu.VMEM((BPW,), jnp.int32),
            pltpu.VMEM((2, CHUNK, W), jnp.int32),
            pltpu.VMEM((2, CHUNK, W), jnp.int32),
            pltpu.VMEM((2, CHUNK, W), jnp.int32),
            pltpu.VMEM((2, CHUNK), jnp.float32),
            pltpu.VMEM((2, CHUNK), jnp.float32),
            pltpu.VMEM((BPW,), jnp.float32),
            pltpu.SemaphoreType.DMA((2,)),
            pltpu.SemaphoreType.DMA((2,)),
        ],
    )(users, pos_items, neg_items, uemb_packed, iemb_packed,
      item_biases_flat)


_PACK_BLK = 32768


def _pack_body(tT_ref, out_ref):
    x = tT_ref[...]
    a = lax.bitcast_convert_type(x[:W, :], jnp.uint32)
    b = lax.bitcast_convert_type(x[W:, :], jnp.uint32)

    def rne(u):
        return (u + jnp.uint32(0x7FFF) + ((u >> 16) & jnp.uint32(1))) >> 16

    packed = rne(a) | (rne(b) << 16)
    out_ref[...] = lax.bitcast_convert_type(packed.T, jnp.int32)


def _pack(table_T):
    n = table_T.shape[1]
    grid = (n + _PACK_BLK - 1) // _PACK_BLK
    return pl.pallas_call(
        _pack_body,
        grid=(grid,),
        in_specs=[pl.BlockSpec((D, _PACK_BLK), lambda i: (0, i))],
        out_specs=pl.BlockSpec((_PACK_BLK, W), lambda i: (i, 0)),
        out_shape=jax.ShapeDtypeStruct((n, W), jnp.int32),
    )(table_T)


def kernel(users, pos_items, neg_items, user_embeddings, item_embeddings,
           user_biases, item_biases):
    del user_biases
    return _bpr_call(users, pos_items, neg_items, _pack(user_embeddings.T),
                     _pack(item_embeddings.T), item_biases.reshape(-1))
